# baseline (speedup 1.0000x reference)
import jax
import jax.numpy as jnp
from jax import lax
from jax.experimental import pallas as pl
from jax.experimental.pallas import tpu as pltpu
from jax.experimental.pallas import tpu_sc as plsc

F = 26
V = 100000
VHA = 49920
VHB = V - VHA
D = 32
B = 4096
NC = 2
NS = 16
NW = NC * NS
LANES = 16


def _tec_body(idx_hbm, tab_hbm, out_hbm, half_a, half_b, idx_v, out_v,
              sem_row, sem_idx, sem_out):
    wid = lax.axis_index("s") * NC + lax.axis_index("c")

    @pl.when(wid < 16)
    def _():
        for f in range(F):
            s = f & 1
            ca = pltpu.async_copy(
                tab_hbm.at[f, wid].at[pl.ds(0, VHA)], half_a, sem_row)
            cb = pltpu.async_copy(
                tab_hbm.at[f, wid].at[pl.ds(VHA, VHB)], half_b, sem_row)
            ca.wait()
            cb.wait()
            ca2 = pltpu.async_copy(
                tab_hbm.at[f, wid + 16].at[pl.ds(0, VHA)], half_a, sem_row)
            cb2 = pltpu.async_copy(
                tab_hbm.at[f, wid + 16].at[pl.ds(VHA, VHB)], half_b, sem_row)
            ca2.wait()
            cb2.wait()
            co = pltpu.async_copy(out_v.at[s], out_hbm.at[f, wid], sem_out)
            co.wait()


@jax.jit
def _gather(idx_t, tab_t):
    mesh = plsc.VectorSubcoreMesh(core_axis_name="c", subcore_axis_name="s")
    run = pl.kernel(
        _tec_body,
        mesh=mesh,
        compiler_params=pltpu.CompilerParams(
            use_tc_tiling_on_sc=True, needs_layout_passes=False
        ),
        out_type=jax.ShapeDtypeStruct((F, D, B), jnp.float32),
        scratch_types=[
            pltpu.VMEM((VHA,), jnp.float32),
            pltpu.VMEM((VHB,), jnp.float32),
            pltpu.VMEM((2, B), jnp.int32),
            pltpu.VMEM((2, B), jnp.float32),
            pltpu.SemaphoreType.DMA,
            pltpu.SemaphoreType.DMA,
            pltpu.SemaphoreType.DMA,
        ],
    )
    return run(idx_t, tab_t)


def kernel(sparse_inputs, tables):
    idx_t = sparse_inputs.astype(jnp.int32).T
    tab_t = tables.transpose(0, 2, 1)
    out_t = _gather(idx_t, tab_t)
    return out_t.transpose(2, 0, 1)


# speedup vs baseline: 1.3246x; 1.3246x over previous
import functools

import jax
import jax.numpy as jnp
from jax import lax
from jax.experimental import pallas as pl
from jax.experimental.pallas import tpu as pltpu
from jax.experimental.pallas import tpu_sc as plsc

F = 26
V = 100000
VHA = 49920
VHB = V - VHA
D = 32
B = 4096
NC = 2
NS = 16
NW = NC * NS
LANES = 16


def _tec_body(idx_hbm, tab_hbm, out_hbm, half_a, half_b, idx_v, out_v,
              sem_row, sem_idx, sem_out):
    wid = lax.axis_index("s") * NC + lax.axis_index("c")

    def pass1(s):
        def body(j, c):
            sl = pl.ds(j * LANES, LANES)
            vidx = jnp.abs(idx_v[s, sl])
            out_v[s, sl] = plsc.load_gather(half_a, [jnp.minimum(vidx, VHA - 1)])
            return c

        lax.fori_loop(0, B // LANES, body, 0)

    def pass2(s):
        def body(j, c):
            sl = pl.ds(j * LANES, LANES)
            vidx = jnp.abs(idx_v[s, sl])
            hi = plsc.load_gather(
                half_b, [jnp.maximum(vidx, VHA) - VHA])
            out_v[s, sl] = jnp.where(vidx < VHA, out_v[s, sl], hi)
            return c

        lax.fori_loop(0, B // LANES, body, 0)

    next_a = pltpu.async_copy(tab_hbm.at[0, wid].at[pl.ds(0, VHA)], half_a, sem_row)
    next_idx = pltpu.async_copy(idx_hbm.at[0], idx_v.at[0], sem_idx)
    out_copies = [None] * F
    for f in range(F):
        s = f & 1
        next_a.wait()
        next_idx.wait()
        copy_b = pltpu.async_copy(
            tab_hbm.at[f, wid].at[pl.ds(VHA, VHB)], half_b, sem_row)
        pass1(s)
        copy_b.wait()
        if f + 1 < F:
            next_a = pltpu.async_copy(
                tab_hbm.at[f + 1, wid].at[pl.ds(0, VHA)], half_a, sem_row)
            next_idx = pltpu.async_copy(
                idx_hbm.at[f + 1], idx_v.at[1 - s], sem_idx)
        pass2(s)
        if f >= 2:
            out_copies[f - 2].wait()
        out_copies[f] = pltpu.async_copy(
            out_v.at[s], out_hbm.at[f, wid], sem_out)
    out_copies[F - 2].wait()
    out_copies[F - 1].wait()


@jax.jit
def _gather(idx_t, tab_t):
    mesh = plsc.VectorSubcoreMesh(core_axis_name="c", subcore_axis_name="s")
    run = pl.kernel(
        _tec_body,
        mesh=mesh,
        compiler_params=pltpu.CompilerParams(
            use_tc_tiling_on_sc=True, needs_layout_passes=False
        ),
        out_type=jax.ShapeDtypeStruct((F, D, B), jnp.float32),
        scratch_types=[
            pltpu.VMEM((VHA,), jnp.float32),
            pltpu.VMEM((VHB,), jnp.float32),
            pltpu.VMEM((2, B), jnp.int32),
            pltpu.VMEM((2, B), jnp.float32),
            pltpu.SemaphoreType.DMA,
            pltpu.SemaphoreType.DMA,
            pltpu.SemaphoreType.DMA,
        ],
    )
    return run(idx_t, tab_t)


def kernel(sparse_inputs, tables):
    idx_t = sparse_inputs.astype(jnp.int32).T
    tab_t = tables.transpose(0, 2, 1)
    out_t = _gather(idx_t, tab_t)
    return out_t.transpose(2, 0, 1)


# speedup vs baseline: 1.3372x; 1.0095x over previous
import jax
import jax.numpy as jnp
from jax import lax
from jax.experimental import pallas as pl
from jax.experimental.pallas import tpu as pltpu
from jax.experimental.pallas import tpu_sc as plsc

F = 26
V = 100000
VHA = 49920
VHB = V - VHA
D = 32
B = 4096
NC = 2
NS = 16
NW = NC * NS
LANES = 16


def _tec_body(idx_hbm, tab_hbm, out_hbm, half_a, half_b, idx_v, out_v,
              sem_row, sem_idx, sem_out):
    wid = lax.axis_index("s") * NC + lax.axis_index("c")

    def pass1(s):
        def body(j, c):
            sl = pl.ds(j * LANES, LANES)
            vidx = jnp.abs(idx_v[s, sl])
            out_v[s, sl] = plsc.load_gather(half_a, [jnp.minimum(vidx, VHA - 1)])
            return c

        lax.fori_loop(0, B // LANES, body, 0)

    def pass2(s):
        def body(j, c):
            sl = pl.ds(j * LANES, LANES)
            vidx = jnp.abs(idx_v[s, sl])
            hi = plsc.load_gather(
                half_b, [jnp.maximum(vidx, VHA) - VHA])
            out_v[s, sl] = jnp.where(vidx < VHA, out_v[s, sl], hi)
            return c

        lax.fori_loop(0, B // LANES, body, 0)

    next_a = pltpu.async_copy(tab_hbm.at[0, wid].at[pl.ds(0, VHA)], half_a, sem_row)
    next_idx = pltpu.async_copy(idx_hbm.at[0], idx_v.at[0], sem_idx)
    out_copies = [None] * F
    for f in range(F):
        s = f & 1
        next_a.wait()
        next_idx.wait()
        copy_b = pltpu.async_copy(
            tab_hbm.at[f, wid].at[pl.ds(VHA, VHB)], half_b, sem_row)
        pass1(s)
        copy_b.wait()
        if f + 1 < F:
            next_a = pltpu.async_copy(
                tab_hbm.at[f + 1, wid].at[pl.ds(0, VHA)], half_a, sem_row)
            next_idx = pltpu.async_copy(
                idx_hbm.at[f + 1], idx_v.at[1 - s], sem_idx)
        pass2(s)
        if f >= 2:
            out_copies[f - 2].wait()
        out_copies[f] = pltpu.async_copy(
            out_v.at[s], out_hbm.at[f, wid], sem_out)
    out_copies[F - 2].wait()
    out_copies[F - 1].wait()


@jax.jit
def _gather(idx_t, tab_t):
    mesh = plsc.VectorSubcoreMesh(core_axis_name="c", subcore_axis_name="s")
    run = pl.kernel(
        _tec_body,
        mesh=mesh,
        compiler_params=pltpu.CompilerParams(
            use_tc_tiling_on_sc=True, needs_layout_passes=False
        ),
        out_type=jax.ShapeDtypeStruct((F, D, B), jnp.float32),
        scratch_types=[
            pltpu.VMEM((VHA,), jnp.float32),
            pltpu.VMEM((VHB,), jnp.float32),
            pltpu.VMEM((2, B), jnp.int32),
            pltpu.VMEM((2, B), jnp.float32),
            pltpu.SemaphoreType.DMA,
            pltpu.SemaphoreType.DMA,
            pltpu.SemaphoreType.DMA,
        ],
    )
    return run(idx_t, tab_t)


def kernel(sparse_inputs, tables):
    idx_t = sparse_inputs.astype(jnp.int32).T
    tab_t = tables.transpose(0, 2, 1)
    out_t = _gather(idx_t, tab_t)
    return out_t.transpose(2, 0, 1)


# speedup vs baseline: 1.3768x; 1.0296x over previous
import jax
import jax.numpy as jnp
from jax import lax
from jax.experimental import pallas as pl
from jax.experimental.pallas import tpu as pltpu
from jax.experimental.pallas import tpu_sc as plsc

F = 26
V = 100000
VHA = 49920
VHB = V - VHA
D = 32
B = 4096
NC = 2
NS = 16
NW = NC * NS
LANES = 16


def _tec_body(idx_hbm, tab_hbm, out_hbm, half_a, half_b, idx_v, out_v,
              sem_row_a, sem_row_b, sem_idx, sem_out):
    wid = lax.axis_index("s") * NC + lax.axis_index("c")

    def pass1(s):
        def body(j, c):
            sl = pl.ds(j * LANES, LANES)
            vidx = jnp.abs(idx_v[s, sl])
            out_v[s, sl] = plsc.load_gather(half_a, [jnp.minimum(vidx, VHA - 1)])
            return c

        lax.fori_loop(0, B // LANES, body, 0)

    def pass2(s):
        def body(j, c):
            sl = pl.ds(j * LANES, LANES)
            vidx = jnp.abs(idx_v[s, sl])
            hi = plsc.load_gather(
                half_b, [jnp.maximum(vidx, VHA) - VHA])
            out_v[s, sl] = jnp.where(vidx < VHA, out_v[s, sl], hi)
            return c

        lax.fori_loop(0, B // LANES, body, 0)

    next_a = pltpu.async_copy(tab_hbm.at[0, wid].at[pl.ds(0, VHA)], half_a, sem_row_a)
    next_idx = pltpu.async_copy(idx_hbm.at[0], idx_v.at[0], sem_idx)
    out_copies = [None] * F
    for f in range(F):
        s = f & 1
        next_a.wait()
        next_idx.wait()
        copy_b = pltpu.async_copy(
            tab_hbm.at[f, wid].at[pl.ds(VHA, VHB)], half_b, sem_row_b)
        if f >= 2:
            out_copies[f - 2].wait()
        pass1(s)
        if f + 1 < F:
            next_a = pltpu.async_copy(
                tab_hbm.at[f + 1, wid].at[pl.ds(0, VHA)], half_a, sem_row_a)
            next_idx = pltpu.async_copy(
                idx_hbm.at[f + 1], idx_v.at[1 - s], sem_idx)
        copy_b.wait()
        pass2(s)
        out_copies[f] = pltpu.async_copy(
            out_v.at[s], out_hbm.at[f, wid], sem_out)
    out_copies[F - 2].wait()
    out_copies[F - 1].wait()


@jax.jit
def _gather(idx_t, tab_t):
    mesh = plsc.VectorSubcoreMesh(core_axis_name="c", subcore_axis_name="s")
    run = pl.kernel(
        _tec_body,
        mesh=mesh,
        compiler_params=pltpu.CompilerParams(
            use_tc_tiling_on_sc=True, needs_layout_passes=False
        ),
        out_type=jax.ShapeDtypeStruct((F, D, B), jnp.float32),
        scratch_types=[
            pltpu.VMEM((VHA,), jnp.float32),
            pltpu.VMEM((VHB,), jnp.float32),
            pltpu.VMEM((2, B), jnp.int32),
            pltpu.VMEM((2, B), jnp.float32),
            pltpu.SemaphoreType.DMA,
            pltpu.SemaphoreType.DMA,
            pltpu.SemaphoreType.DMA,
            pltpu.SemaphoreType.DMA,
        ],
    )
    return run(idx_t, tab_t)


def kernel(sparse_inputs, tables):
    idx_t = sparse_inputs.astype(jnp.int32).T
    tab_t = tables.transpose(0, 2, 1)
    out_t = _gather(idx_t, tab_t)
    return out_t.transpose(2, 0, 1)
